# Initial kernel scaffold; baseline (speedup 1.0000x reference)
import jax
import jax.numpy as jnp
from jax.experimental import pallas as pl


def kernel(x, edge_index, edge_weight, W1, W2):
    raise NotImplementedError("write your pallas kernel here")


# speedup vs baseline: 4.0830x; 4.0830x over previous
import functools

import jax
import jax.numpy as jnp
from jax import lax
from jax.experimental import pallas as pl
from jax.experimental.pallas import tpu as pltpu
from jax.experimental.pallas import tpu_sc as plsc

NC = 2
NS = 16
LANES = 16


def _elu(v):
    return jnp.where(v > 0, v, jnp.exp(v) - 1.0)


def _mm_body(x_ref, w_ref, o_ref):
    o_ref[...] = jnp.dot(x_ref[...], w_ref[...],
                         preferred_element_type=jnp.float32)


def _fuse_mm_body(p0_ref, p1_ref, w_ref, o_ref):
    h = _elu(p0_ref[...] + p1_ref[...])
    o_ref[...] = jnp.dot(h, w_ref[...], preferred_element_type=jnp.float32)


def _final_body(q0_ref, q1_ref, o_ref):
    m = _elu(q0_ref[...] + q1_ref[...])
    n = jnp.sqrt(jnp.sum(m * m, axis=1, keepdims=True))
    o_ref[...] = m / jnp.maximum(n, 1e-12)


def _tc_matmul(x, w, bm=1000):
    m, k = x.shape
    h = w.shape[1]
    return pl.pallas_call(
        _mm_body,
        grid=(m // bm,),
        in_specs=[pl.BlockSpec((bm, k), lambda i: (i, 0)),
                  pl.BlockSpec((k, h), lambda i: (0, 0))],
        out_specs=pl.BlockSpec((bm, h), lambda i: (i, 0)),
        out_shape=jax.ShapeDtypeStruct((m, h), jnp.float32),
    )(x, w)


def _tc_fuse_matmul(p0, p1, w, bm=1000):
    m, k = p0.shape
    h = w.shape[1]
    return pl.pallas_call(
        _fuse_mm_body,
        grid=(m // bm,),
        in_specs=[pl.BlockSpec((bm, k), lambda i: (i, 0)),
                  pl.BlockSpec((bm, k), lambda i: (i, 0)),
                  pl.BlockSpec((k, h), lambda i: (0, 0))],
        out_specs=pl.BlockSpec((bm, h), lambda i: (i, 0)),
        out_shape=jax.ShapeDtypeStruct((m, h), jnp.float32),
    )(p0, p1, w)


def _tc_final(q0, q1, bm=1000):
    m, h = q0.shape
    return pl.pallas_call(
        _final_body,
        grid=(m // bm,),
        in_specs=[pl.BlockSpec((bm, h), lambda i: (i, 0)),
                  pl.BlockSpec((bm, h), lambda i: (i, 0))],
        out_specs=pl.BlockSpec((bm, h), lambda i: (i, 0)),
        out_shape=jax.ShapeDtypeStruct((m, h), jnp.float32),
    )(q0, q1)


@functools.lru_cache(maxsize=None)
def _make_spmm(n_nodes, n_edges, d, chunk=80):
    n_tiles = NC * NS
    assert n_edges % (n_tiles * chunk) == 0
    e_per_tile = n_edges // n_tiles
    n_chunks = e_per_tile // chunk
    n_pad = ((n_nodes + 8 * NS - 1) // (8 * NS)) * (8 * NS)
    rpt = n_pad // NS
    dk = d // LANES
    assert d % LANES == 0

    mesh = plsc.VectorSubcoreMesh(core_axis_name="c", subcore_axis_name="s",
                                  num_cores=NC, num_subcores=NS)

    def body(support, src, dst, w, out, acc, src_v, dst_v, w_v, rows_v, zbuf):
        c = lax.axis_index("c")
        s = lax.axis_index("s")

        zero16 = jnp.zeros((LANES,), jnp.float32)

        def zrow(r, carry):
            for k in range(dk):
                zbuf[r, pl.ds(k * LANES, LANES)] = zero16
            return carry

        lax.fori_loop(0, rpt, zrow, 0)
        pltpu.sync_copy(zbuf, acc.at[pl.ds(s * rpt, rpt)])
        plsc.subcore_barrier()

        base = (c * NS + s) * e_per_tile

        def chunk_body(j, carry):
            off = base + j * chunk
            pltpu.sync_copy(src.at[pl.ds(off, chunk)], src_v)
            pltpu.sync_copy(dst.at[pl.ds(off, chunk)], dst_v)
            pltpu.sync_copy(w.at[pl.ds(off, chunk)], w_v)
            pltpu.sync_copy(support.at[src_v], rows_v)

            def mul(g, cc):
                wv = w_v[pl.ds(g * LANES, LANES)]
                for l in range(LANES):
                    ws = jnp.broadcast_to(wv[l], (LANES,))
                    e = g * LANES + l
                    for k in range(dk):
                        sl = (e, pl.ds(k * LANES, LANES))
                        rows_v[sl] = rows_v[sl] * ws
                return cc

            lax.fori_loop(0, chunk // LANES, mul, 0)
            pltpu.sync_copy(rows_v, acc.at[dst_v], add=True)
            return carry

        lax.fori_loop(0, n_chunks, chunk_body, 0)
        plsc.subcore_barrier()

        r0 = s * rpt
        pltpu.sync_copy(acc.at[pl.ds(r0, rpt)],
                        out.at[pl.ds(c * n_pad + r0, rpt)])

    return pl.kernel(
        body,
        out_type=jax.ShapeDtypeStruct((NC * n_pad, d), jnp.float32),
        mesh=mesh,
        compiler_params=pltpu.CompilerParams(use_tc_tiling_on_sc=False),
        scratch_types=[
            pltpu.VMEM_SHARED((n_pad, d), jnp.float32),
            pltpu.VMEM((chunk,), jnp.int32),
            pltpu.VMEM((chunk,), jnp.int32),
            pltpu.VMEM((chunk,), jnp.float32),
            pltpu.VMEM((chunk, d), jnp.float32),
            pltpu.VMEM((rpt, d), jnp.float32),
        ],
    )


def _spmm(support, src, dst, w):
    n, d = support.shape
    e = src.shape[0]
    fn = _make_spmm(n, e, d)
    partials = fn(support, src, dst, w)
    n_pad = partials.shape[0] // NC
    return partials[:n], partials[n_pad:n_pad + n]


def kernel(x, edge_index, edge_weight, W1, W2):
    src = edge_index[0].astype(jnp.int32)
    dst = edge_index[1].astype(jnp.int32)
    w = edge_weight.astype(jnp.float32)

    support1 = _tc_matmul(x, W1)
    p0, p1 = _spmm(support1, src, dst, w)
    support2 = _tc_fuse_matmul(p0, p1, W2)
    q0, q1 = _spmm(support2, src, dst, w)
    return _tc_final(q0, q1)


# speedup vs baseline: 8.6625x; 2.1216x over previous
import functools

import jax
import jax.numpy as jnp
from jax import lax
from jax.experimental import pallas as pl
from jax.experimental.pallas import tpu as pltpu
from jax.experimental.pallas import tpu_sc as plsc

NC = 2
NS = 16
LANES = 16


def _elu(v):
    return jnp.where(v > 0, v, jnp.exp(v) - 1.0)


def _mm_body(x_ref, w_ref, o_ref):
    o_ref[...] = jnp.dot(x_ref[...], w_ref[...],
                         preferred_element_type=jnp.float32)


def _fuse_mm_body(p0_ref, p1_ref, w_ref, o_ref):
    h = _elu(p0_ref[...] + p1_ref[...])
    o_ref[...] = jnp.dot(h, w_ref[...], preferred_element_type=jnp.float32)


def _final_body(q0_ref, q1_ref, o_ref):
    m = _elu(q0_ref[...] + q1_ref[...])
    n = jnp.sqrt(jnp.sum(m * m, axis=1, keepdims=True))
    o_ref[...] = m / jnp.maximum(n, 1e-12)


def _tc_matmul(x, w, bm=1000):
    m, k = x.shape
    h = w.shape[1]
    return pl.pallas_call(
        _mm_body,
        grid=(m // bm,),
        in_specs=[pl.BlockSpec((bm, k), lambda i: (i, 0)),
                  pl.BlockSpec((k, h), lambda i: (0, 0))],
        out_specs=pl.BlockSpec((bm, h), lambda i: (i, 0)),
        out_shape=jax.ShapeDtypeStruct((m, h), jnp.float32),
    )(x, w)


def _tc_fuse_matmul(p0, p1, w, bm=1000):
    m, k = p0.shape
    h = w.shape[1]
    return pl.pallas_call(
        _fuse_mm_body,
        grid=(m // bm,),
        in_specs=[pl.BlockSpec((bm, k), lambda i: (i, 0)),
                  pl.BlockSpec((bm, k), lambda i: (i, 0)),
                  pl.BlockSpec((k, h), lambda i: (0, 0))],
        out_specs=pl.BlockSpec((bm, h), lambda i: (i, 0)),
        out_shape=jax.ShapeDtypeStruct((m, h), jnp.float32),
    )(p0, p1, w)


def _tc_final(q0, q1, bm=1000):
    m, h = q0.shape
    return pl.pallas_call(
        _final_body,
        grid=(m // bm,),
        in_specs=[pl.BlockSpec((bm, h), lambda i: (i, 0)),
                  pl.BlockSpec((bm, h), lambda i: (i, 0))],
        out_specs=pl.BlockSpec((bm, h), lambda i: (i, 0)),
        out_shape=jax.ShapeDtypeStruct((m, h), jnp.float32),
    )(q0, q1)


@functools.lru_cache(maxsize=None)
def _make_spmm(n_nodes, n_chunks, d, chunk):
    assert n_chunks % 2 == 0
    n_pad = ((n_nodes + 8 * NS - 1) // (8 * NS)) * (8 * NS)
    rpt = n_pad // NS
    dk = d // LANES
    assert d % LANES == 0

    mesh = plsc.VectorSubcoreMesh(core_axis_name="c", subcore_axis_name="s",
                                  num_cores=NC, num_subcores=NS)

    def body(support, ei, w, out, acc, sd, wt, r0, r1, zbuf,
             gs0, gs1, ss0, ss1):
        c = lax.axis_index("c")
        s = lax.axis_index("s")
        t = c * NS + s

        ld_sd = pltpu.async_copy(ei.at[t], sd, gs0)
        ld_wt = pltpu.async_copy(w.at[t], wt, gs1)

        zero16 = jnp.zeros((LANES,), jnp.float32)

        def zrow(r, carry):
            for k in range(dk):
                zbuf[r, pl.ds(k * LANES, LANES)] = zero16
            return carry

        lax.fori_loop(0, rpt, zrow, 0)
        pltpu.sync_copy(zbuf, acc.at[pl.ds(s * rpt, rpt)])
        ld_sd.wait()
        ld_wt.wait()
        plsc.subcore_barrier()

        def scale(jc, rows):
            def mul(g, cc):
                wv = wt[jc, pl.ds(g * LANES, LANES)]
                for l in range(LANES):
                    ws = jnp.broadcast_to(wv[l], (LANES,))
                    e = g * LANES + l
                    for k in range(dk):
                        sl = (e, pl.ds(k * LANES, LANES))
                        rows[sl] = rows[sl] * ws
                return cc

            lax.fori_loop(0, chunk // LANES, mul, 0)

        def pair(i, carry):
            j0 = 2 * i
            j1 = 2 * i + 1
            @pl.when(i > 0)
            def _():
                pltpu.make_async_copy(r0, acc.at[sd.at[2]], ss0).wait()
            g0 = pltpu.async_copy(support.at[sd.at[2 * j0]], r0, gs0)

            @pl.when(i > 0)
            def _():
                pltpu.make_async_copy(r1, acc.at[sd.at[2]], ss1).wait()
            g0.wait()
            g1 = pltpu.async_copy(support.at[sd.at[2 * j1]], r1, gs1)
            scale(j0, r0)
            pltpu.async_copy(r0, acc.at[sd.at[2 * j0 + 1]], ss0, add=True)
            g1.wait()
            scale(j1, r1)
            pltpu.async_copy(r1, acc.at[sd.at[2 * j1 + 1]], ss1, add=True)
            return carry

        lax.fori_loop(0, n_chunks // 2, pair, 0)
        pltpu.make_async_copy(r0, acc.at[sd.at[2]], ss0).wait()
        pltpu.make_async_copy(r1, acc.at[sd.at[2]], ss1).wait()
        plsc.subcore_barrier()

        rr = s * rpt
        pltpu.sync_copy(acc.at[pl.ds(rr, rpt)],
                        out.at[pl.ds(c * n_pad + rr, rpt)])

    return pl.kernel(
        body,
        out_type=jax.ShapeDtypeStruct((NC * n_pad, d), jnp.float32),
        mesh=mesh,
        compiler_params=pltpu.CompilerParams(use_tc_tiling_on_sc=False),
        scratch_types=[
            pltpu.VMEM_SHARED((n_pad, d), jnp.float32),
            pltpu.VMEM((2 * n_chunks, chunk), jnp.int32),
            pltpu.VMEM((n_chunks, chunk), jnp.float32),
            pltpu.VMEM((chunk, d), jnp.float32),
            pltpu.VMEM((chunk, d), jnp.float32),
            pltpu.VMEM((rpt, d), jnp.float32),
            pltpu.SemaphoreType.DMA,
            pltpu.SemaphoreType.DMA,
            pltpu.SemaphoreType.DMA,
            pltpu.SemaphoreType.DMA,
        ],
    )


_CHUNK = 128


def _prep_edges(src, dst, w, n_nodes):
    e = src.shape[0]
    t = NC * NS
    per_tile = -(-e // (t * 2 * _CHUNK)) * (2 * _CHUNK)
    e_pad = per_tile * t
    pad = e_pad - e
    if pad:
        pidx = jnp.arange(pad, dtype=jnp.int32) % jnp.int32(n_nodes)
        src = jnp.concatenate([src, pidx])
        dst = jnp.concatenate([dst, pidx])
        w = jnp.concatenate([w, jnp.zeros((pad,), jnp.float32)])
    nc = per_tile // _CHUNK
    srcp = src.reshape(t, nc, 1, _CHUNK)
    dstp = dst.reshape(t, nc, 1, _CHUNK)
    ei = jnp.concatenate([srcp, dstp], axis=2).reshape(t, 2 * nc, _CHUNK)
    wt = w.reshape(t, nc, _CHUNK)
    return ei, wt, nc


def _spmm(support, ei, wt, nc):
    n, d = support.shape
    fn = _make_spmm(n, nc, d, _CHUNK)
    partials = fn(support, ei, wt)
    n_pad = partials.shape[0] // NC
    return partials[:n], partials[n_pad:n_pad + n]


def kernel(x, edge_index, edge_weight, W1, W2):
    src = edge_index[0].astype(jnp.int32)
    dst = edge_index[1].astype(jnp.int32)
    w = edge_weight.astype(jnp.float32)
    ei, wt, nc = _prep_edges(src, dst, w, x.shape[0])

    support1 = _tc_matmul(x, W1)
    p0, p1 = _spmm(support1, ei, wt, nc)
    support2 = _tc_fuse_matmul(p0, p1, W2)
    q0, q1 = _spmm(support2, ei, wt, nc)
    return _tc_final(q0, q1)


# speedup vs baseline: 10.9105x; 1.2595x over previous
import functools

import jax
import jax.numpy as jnp
from jax import lax
from jax.experimental import pallas as pl
from jax.experimental.pallas import tpu as pltpu
from jax.experimental.pallas import tpu_sc as plsc

NC = 2
NS = 16
LANES = 16


def _elu(v):
    return jnp.where(v > 0, v, jnp.exp(v) - 1.0)


def _mm_body(x_ref, w_ref, o_ref):
    o_ref[...] = jnp.dot(x_ref[...], w_ref[...],
                         preferred_element_type=jnp.float32)


def _fuse_mm_body(p0_ref, p1_ref, w_ref, o_ref):
    h = _elu(p0_ref[...] + p1_ref[...])
    o_ref[...] = jnp.dot(h, w_ref[...], preferred_element_type=jnp.float32)


def _final_body(q0_ref, q1_ref, o_ref):
    m = _elu(q0_ref[...] + q1_ref[...])
    n = jnp.sqrt(jnp.sum(m * m, axis=1, keepdims=True))
    o_ref[...] = m / jnp.maximum(n, 1e-12)


def _tc_matmul(x, w, bm=1000):
    m, k = x.shape
    h = w.shape[1]
    return pl.pallas_call(
        _mm_body,
        grid=(m // bm,),
        in_specs=[pl.BlockSpec((bm, k), lambda i: (i, 0)),
                  pl.BlockSpec((k, h), lambda i: (0, 0))],
        out_specs=pl.BlockSpec((bm, h), lambda i: (i, 0)),
        out_shape=jax.ShapeDtypeStruct((m, h), jnp.float32),
    )(x, w)


def _tc_fuse_matmul(p0, p1, w, bm=1000):
    m, k = p0.shape
    h = w.shape[1]
    return pl.pallas_call(
        _fuse_mm_body,
        grid=(m // bm,),
        in_specs=[pl.BlockSpec((bm, k), lambda i: (i, 0)),
                  pl.BlockSpec((bm, k), lambda i: (i, 0)),
                  pl.BlockSpec((k, h), lambda i: (0, 0))],
        out_specs=pl.BlockSpec((bm, h), lambda i: (i, 0)),
        out_shape=jax.ShapeDtypeStruct((m, h), jnp.float32),
    )(p0, p1, w)


def _tc_final(q0, q1, bm=1000):
    m, h = q0.shape
    return pl.pallas_call(
        _final_body,
        grid=(m // bm,),
        in_specs=[pl.BlockSpec((bm, h), lambda i: (i, 0)),
                  pl.BlockSpec((bm, h), lambda i: (i, 0))],
        out_specs=pl.BlockSpec((bm, h), lambda i: (i, 0)),
        out_shape=jax.ShapeDtypeStruct((m, h), jnp.float32),
    )(q0, q1)


@functools.lru_cache(maxsize=None)
def _make_spmm(n_nodes, n_chunks, d, chunk):
    assert n_chunks % 2 == 0
    n_pad = ((n_nodes + 8 * NS - 1) // (8 * NS)) * (8 * NS)
    rpt = n_pad // NS
    dk = d // LANES
    assert d % LANES == 0

    mesh = plsc.VectorSubcoreMesh(core_axis_name="c", subcore_axis_name="s",
                                  num_cores=NC, num_subcores=NS)

    def body(support, ei, w, zeros, out, acc, sd, wt, r0, r1, r2, r3,
             gs0, gs1, gs2, gs3, ss0, ss1, ss2, ss3):
        c = lax.axis_index("c")
        s = lax.axis_index("s")
        t = c * NS + s

        ld_sd = pltpu.async_copy(ei.at[t], sd, gs0)
        ld_wt = pltpu.async_copy(w.at[t], wt, gs1)

        pltpu.sync_copy(zeros.at[pl.ds(s * rpt, rpt)],
                        acc.at[pl.ds(s * rpt, rpt)])
        ld_sd.wait()
        ld_wt.wait()
        plsc.subcore_barrier()

        def scale(jc, rows):
            def mul(g, cc):
                wv = wt[jc, pl.ds(g * LANES, LANES)]
                for l in range(LANES):
                    ws = jnp.broadcast_to(wv[l], (LANES,))
                    e = g * LANES + l
                    for k in range(dk):
                        sl = (e, pl.ds(k * LANES, LANES))
                        rows[sl] = rows[sl] * ws
                return cc

            lax.fori_loop(0, chunk // LANES, mul, 0)

        rows = (r0, r1, r2, r3)
        gsem = (gs0, gs1, gs2, gs3)
        ssem = (ss0, ss1, ss2, ss3)
        nbuf = 4
        assert n_chunks % nbuf == 0 and n_chunks >= 2 * nbuf

        def wait_gather(b):
            pltpu.make_async_copy(support.at[sd.at[0]], rows[b],
                                  gsem[b]).wait()

        def wait_scatter(b):
            pltpu.make_async_copy(rows[b], acc.at[sd.at[2]], ssem[b]).wait()

        def step(j, b, sswait=True, issue=True):
            wait_gather(b)
            scale(j, rows[b])
            pltpu.async_copy(rows[b], acc.at[sd.at[2 * j + 1]],
                             ssem[b], add=True)
            nb = (b + nbuf - 1) % nbuf
            if sswait:
                wait_scatter(nb)
            if issue:
                pltpu.async_copy(support.at[sd.at[2 * (j + nbuf - 1)]],
                                 rows[nb], gsem[nb])

        for b in range(nbuf - 1):
            pltpu.async_copy(support.at[sd.at[2 * b]], rows[b], gsem[b])
        for b in range(nbuf):
            step(b, b, sswait=(b > 0))

        def quad(i, carry):
            for b in range(nbuf):
                step(nbuf * i + b, b)
            return carry

        lax.fori_loop(1, n_chunks // nbuf - 1, quad, 0)
        for b in range(nbuf):
            step(n_chunks - nbuf + b, b, issue=(b == 0))
        wait_scatter((n_chunks - 1) % nbuf)
        plsc.subcore_barrier()

        rr = s * rpt
        pltpu.sync_copy(acc.at[pl.ds(rr, rpt)],
                        out.at[pl.ds(c * n_pad + rr, rpt)])

    return pl.kernel(
        body,
        out_type=jax.ShapeDtypeStruct((NC * n_pad, d), jnp.float32),
        mesh=mesh,
        compiler_params=pltpu.CompilerParams(use_tc_tiling_on_sc=False),
        scratch_types=[
            pltpu.VMEM_SHARED((n_pad, d), jnp.float32),
            pltpu.VMEM((2 * n_chunks, chunk), jnp.int32),
            pltpu.VMEM((n_chunks, chunk), jnp.float32),
            pltpu.VMEM((chunk, d), jnp.float32),
            pltpu.VMEM((chunk, d), jnp.float32),
            pltpu.VMEM((chunk, d), jnp.float32),
            pltpu.VMEM((chunk, d), jnp.float32),
            pltpu.SemaphoreType.DMA,
            pltpu.SemaphoreType.DMA,
            pltpu.SemaphoreType.DMA,
            pltpu.SemaphoreType.DMA,
            pltpu.SemaphoreType.DMA,
            pltpu.SemaphoreType.DMA,
            pltpu.SemaphoreType.DMA,
            pltpu.SemaphoreType.DMA,
        ],
    )


_CHUNK = 128


def _prep_edges(src, dst, w, n_nodes):
    e = src.shape[0]
    t = NC * NS
    per_tile = -(-e // (t * 2 * _CHUNK)) * (2 * _CHUNK)
    e_pad = per_tile * t
    pad = e_pad - e
    if pad:
        pidx = jnp.arange(pad, dtype=jnp.int32) % jnp.int32(n_nodes)
        src = jnp.concatenate([src, pidx])
        dst = jnp.concatenate([dst, pidx])
        w = jnp.concatenate([w, jnp.zeros((pad,), jnp.float32)])
    nc = per_tile // _CHUNK
    srcp = src.reshape(t, nc, 1, _CHUNK)
    dstp = dst.reshape(t, nc, 1, _CHUNK)
    ei = jnp.concatenate([srcp, dstp], axis=2).reshape(t, 2 * nc, _CHUNK)
    wt = w.reshape(t, nc, _CHUNK)
    return ei, wt, nc


def _spmm(support, ei, wt, nc):
    n, d = support.shape
    n_pad = ((n + 8 * NS - 1) // (8 * NS)) * (8 * NS)
    zeros = jnp.zeros((n_pad, d), jnp.float32)
    fn = _make_spmm(n, nc, d, _CHUNK)
    partials = fn(support, ei, wt, zeros)
    return partials[:n], partials[n_pad:n_pad + n]


def kernel(x, edge_index, edge_weight, W1, W2):
    src = edge_index[0].astype(jnp.int32)
    dst = edge_index[1].astype(jnp.int32)
    w = edge_weight.astype(jnp.float32)
    ei, wt, nc = _prep_edges(src, dst, w, x.shape[0])

    support1 = _tc_matmul(x, W1)
    p0, p1 = _spmm(support1, ei, wt, nc)
    support2 = _tc_fuse_matmul(p0, p1, W2)
    q0, q1 = _spmm(support2, ei, wt, nc)
    return _tc_final(q0, q1)


# speedup vs baseline: 11.0415x; 1.0120x over previous
import functools

import jax
import jax.numpy as jnp
from jax import lax
from jax.experimental import pallas as pl
from jax.experimental.pallas import tpu as pltpu
from jax.experimental.pallas import tpu_sc as plsc

NC = 2
NS = 16
LANES = 16


def _elu(v):
    return jnp.where(v > 0, v, jnp.exp(v) - 1.0)


def _mm_body(x_ref, w_ref, o_ref):
    o_ref[...] = jnp.dot(x_ref[...], w_ref[...],
                         preferred_element_type=jnp.float32)


def _fuse_mm_body(p0_ref, p1_ref, w_ref, o_ref):
    h = _elu(p0_ref[...] + p1_ref[...])
    o_ref[...] = jnp.dot(h, w_ref[...], preferred_element_type=jnp.float32)


def _final_body(q0_ref, q1_ref, o_ref):
    m = _elu(q0_ref[...] + q1_ref[...])
    n = jnp.sqrt(jnp.sum(m * m, axis=1, keepdims=True))
    o_ref[...] = m / jnp.maximum(n, 1e-12)


def _tc_matmul(x, w, bm=1000):
    m, k = x.shape
    h = w.shape[1]
    return pl.pallas_call(
        _mm_body,
        grid=(m // bm,),
        in_specs=[pl.BlockSpec((bm, k), lambda i: (i, 0)),
                  pl.BlockSpec((k, h), lambda i: (0, 0))],
        out_specs=pl.BlockSpec((bm, h), lambda i: (i, 0)),
        out_shape=jax.ShapeDtypeStruct((m, h), jnp.float32),
    )(x, w)


def _tc_fuse_matmul(p0, p1, w, bm=1000):
    m, k = p0.shape
    h = w.shape[1]
    return pl.pallas_call(
        _fuse_mm_body,
        grid=(m // bm,),
        in_specs=[pl.BlockSpec((bm, k), lambda i: (i, 0)),
                  pl.BlockSpec((bm, k), lambda i: (i, 0)),
                  pl.BlockSpec((k, h), lambda i: (0, 0))],
        out_specs=pl.BlockSpec((bm, h), lambda i: (i, 0)),
        out_shape=jax.ShapeDtypeStruct((m, h), jnp.float32),
    )(p0, p1, w)


def _tc_final(q0, q1, bm=1000):
    m, h = q0.shape
    return pl.pallas_call(
        _final_body,
        grid=(m // bm,),
        in_specs=[pl.BlockSpec((bm, h), lambda i: (i, 0)),
                  pl.BlockSpec((bm, h), lambda i: (i, 0))],
        out_specs=pl.BlockSpec((bm, h), lambda i: (i, 0)),
        out_shape=jax.ShapeDtypeStruct((m, h), jnp.float32),
    )(q0, q1)


@functools.lru_cache(maxsize=None)
def _make_spmm(n_nodes, n_chunks, d, chunk):
    assert n_chunks % 2 == 0
    n_pad = ((n_nodes + 8 * NS - 1) // (8 * NS)) * (8 * NS)
    rpt = n_pad // NS
    dk = d // LANES
    assert d % LANES == 0

    mesh = plsc.VectorSubcoreMesh(core_axis_name="c", subcore_axis_name="s",
                                  num_cores=NC, num_subcores=NS)

    def body(support, es, ed, w, zeros, out, acc, ssrc, sdst, wt,
             r0, r1, r2, r3, gs0, gs1, gs2, gs3, ss0, ss1, ss2, ss3):
        c = lax.axis_index("c")
        s = lax.axis_index("s")
        t = c * NS + s

        ld_src = pltpu.async_copy(es.at[t], ssrc, gs0)
        ld_dst = pltpu.async_copy(ed.at[t], sdst, gs1)
        ld_wt = pltpu.async_copy(w.at[t], wt, gs2)

        pltpu.sync_copy(zeros.at[pl.ds(s * rpt, rpt)],
                        acc.at[pl.ds(s * rpt, rpt)])
        ld_src.wait()
        ld_dst.wait()
        ld_wt.wait()
        plsc.subcore_barrier()

        def scale(jc, rows):
            def mul(g, cc):
                wv = wt[jc, pl.ds(g * LANES, LANES)]
                for l in range(LANES):
                    ws = jnp.broadcast_to(wv[l], (LANES,))
                    e = g * LANES + l
                    for k in range(dk):
                        sl = (e, pl.ds(k * LANES, LANES))
                        rows[sl] = rows[sl] * ws
                return cc

            lax.fori_loop(0, chunk // LANES, mul, 0)

        rows = (r0, r1, r2, r3)
        gsem = (gs0, gs1, gs2, gs3)
        ssem = (ss0, ss1, ss2, ss3)
        nbuf = 4
        assert n_chunks % nbuf == 0 and n_chunks >= 2 * nbuf

        def wait_gather(b):
            pltpu.make_async_copy(support.at[ssrc.at[0]], rows[b],
                                  gsem[b]).wait()

        def wait_scatter(b):
            pltpu.make_async_copy(rows[b], acc.at[sdst.at[0]],
                                  ssem[b]).wait()

        def step(j, b, sswait=True, issue=True):
            wait_gather(b)
            scale(j, rows[b])
            pltpu.async_copy(rows[b], acc.at[sdst.at[j]], ssem[b], add=True)
            nb = (b + nbuf - 1) % nbuf
            if sswait:
                wait_scatter(nb)
            if issue:
                pltpu.async_copy(support.at[ssrc.at[j + nbuf - 1]],
                                 rows[nb], gsem[nb])

        for b in range(nbuf - 1):
            pltpu.async_copy(support.at[ssrc.at[b]], rows[b], gsem[b])
        for b in range(nbuf):
            step(b, b, sswait=(b > 0))

        def quad(i, carry):
            for b in range(nbuf):
                step(nbuf * i + b, b)
            return carry

        lax.fori_loop(1, n_chunks // nbuf - 1, quad, 0)
        for b in range(nbuf):
            step(n_chunks - nbuf + b, b, issue=(b == 0))
        wait_scatter((n_chunks - 1) % nbuf)
        plsc.subcore_barrier()

        rr = s * rpt
        pltpu.sync_copy(acc.at[pl.ds(rr, rpt)],
                        out.at[pl.ds(c * n_pad + rr, rpt)])

    return pl.kernel(
        body,
        out_type=jax.ShapeDtypeStruct((NC * n_pad, d), jnp.float32),
        mesh=mesh,
        compiler_params=pltpu.CompilerParams(use_tc_tiling_on_sc=False),
        scratch_types=[
            pltpu.VMEM_SHARED((n_pad, d), jnp.float32),
            pltpu.VMEM((n_chunks, chunk), jnp.int32),
            pltpu.VMEM((n_chunks, chunk), jnp.int32),
            pltpu.VMEM((n_chunks, chunk), jnp.float32),
            pltpu.VMEM((chunk, d), jnp.float32),
            pltpu.VMEM((chunk, d), jnp.float32),
            pltpu.VMEM((chunk, d), jnp.float32),
            pltpu.VMEM((chunk, d), jnp.float32),
            pltpu.SemaphoreType.DMA,
            pltpu.SemaphoreType.DMA,
            pltpu.SemaphoreType.DMA,
            pltpu.SemaphoreType.DMA,
            pltpu.SemaphoreType.DMA,
            pltpu.SemaphoreType.DMA,
            pltpu.SemaphoreType.DMA,
            pltpu.SemaphoreType.DMA,
        ],
    )


_CHUNK = 128


def _prep_edges(src, dst, w, n_nodes):
    e = src.shape[0]
    t = NC * NS
    per_tile = -(-e // (t * 2 * _CHUNK)) * (2 * _CHUNK)
    e_pad = per_tile * t
    pad = e_pad - e
    if pad:
        pidx = jnp.arange(pad, dtype=jnp.int32) % jnp.int32(n_nodes)
        src = jnp.concatenate([src, pidx])
        dst = jnp.concatenate([dst, pidx])
        w = jnp.concatenate([w, jnp.zeros((pad,), jnp.float32)])
    nc = per_tile // _CHUNK
    es = src.reshape(t, nc, _CHUNK)
    ed = dst.reshape(t, nc, _CHUNK)
    wt = w.reshape(t, nc, _CHUNK)
    return es, ed, wt, nc


def _spmm(support, es, ed, wt, nc):
    n, d = support.shape
    n_pad = ((n + 8 * NS - 1) // (8 * NS)) * (8 * NS)
    zeros = jnp.zeros((n_pad, d), jnp.float32)
    fn = _make_spmm(n, nc, d, _CHUNK)
    partials = fn(support, es, ed, wt, zeros)
    return partials[:n], partials[n_pad:n_pad + n]


def kernel(x, edge_index, edge_weight, W1, W2):
    src = edge_index[0].astype(jnp.int32)
    dst = edge_index[1].astype(jnp.int32)
    w = edge_weight.astype(jnp.float32)
    es, ed, wt, nc = _prep_edges(src, dst, w, x.shape[0])

    support1 = _tc_matmul(x, W1)
    p0, p1 = _spmm(support1, es, ed, wt, nc)
    support2 = _tc_fuse_matmul(p0, p1, W2)
    q0, q1 = _spmm(support2, es, ed, wt, nc)
    return _tc_final(q0, q1)
